# baseline (speedup 1.0000x reference)
import functools

import jax
import jax.numpy as jnp
from jax import lax
from jax.experimental import pallas as pl
from jax.experimental.pallas import tpu as pltpu
from jax.experimental.pallas import tpu_sc as plsc

N = 10000
NP = 10240
K = 10
KPAD = 16


_KNN_R = 256


_SUBR = 16
_NG = NP // 128


def _knn_body(q_ref, c_ref, sqq_ref, sqc_ref, o_ref):
    pid = pl.program_id(0)
    lane = lax.broadcasted_iota(jnp.int32, (_SUBR, 128), 1)
    BIGF = jnp.float32(4e30)
    BIGI = jnp.int32(NP)

    def sub_body(s, carry):
        rows = (pid * _KNN_R + s * _SUBR
                + lax.broadcasted_iota(jnp.int32, (_SUBR, 1), 0))
        qs = q_ref[pl.ds(s * _SUBR, _SUBR), :]
        sqs = sqq_ref[pl.ds(s * _SUBR, _SUBR), :]
        V1 = V2 = V3 = V4 = jnp.full((_SUBR, 128), BIGF, jnp.float32)
        I1 = I2 = I3 = I4 = jnp.full((_SUBR, 128), BIGI, jnp.int32)
        for g in range(_NG):
            dg = jnp.dot(qs, c_ref[:, g * 128:(g + 1) * 128],
                         preferred_element_type=jnp.float32)
            d2 = (sqs + sqc_ref[0:1, g * 128:(g + 1) * 128]) - 2.0 * dg
            colg = lane + (g * 128)
            d2 = jnp.where(colg == rows, BIGF, d2)
            b1 = d2 < V1
            b2 = d2 < V2
            b3 = d2 < V3
            b4 = d2 < V4
            V4 = jnp.where(b4, jnp.where(b3, V3, d2), V4)
            I4 = jnp.where(b4, jnp.where(b3, I3, colg), I4)
            V3 = jnp.where(b3, jnp.where(b2, V2, d2), V3)
            I3 = jnp.where(b3, jnp.where(b2, I2, colg), I3)
            V2 = jnp.where(b2, jnp.where(b1, V1, d2), V2)
            I2 = jnp.where(b2, jnp.where(b1, I1, colg), I2)
            V1 = jnp.where(b1, d2, V1)
            I1 = jnp.where(b1, colg, I1)
        VV = jnp.concatenate([V1, V2, V3, V4], axis=1)
        II = jnp.concatenate([I1, I2, I3, I4], axis=1)
        m = idx = None
        for t in range(K):
            m = jnp.min(VV, axis=1)
            idx = jnp.min(jnp.where(VV == m[:, None], II, BIGI), axis=1)
            o_ref[pl.ds(s * _SUBR, _SUBR), pl.ds(t, 1)] = idx[:, None]
            VV = jnp.where((VV == m[:, None]) & (II == idx[:, None]), BIGF, VV)
        bad = (V4 < m[:, None]) | ((V4 == m[:, None]) & (I4 <= idx[:, None]))
        flag = jnp.any(bad)

        @pl.when(flag)
        def _fallback():
            df = jnp.dot(qs, c_ref[...], preferred_element_type=jnp.float32)
            d2f = (sqs + sqc_ref[0:1, :]) - 2.0 * df
            colf = lax.broadcasted_iota(jnp.int32, (_SUBR, NP), 1)
            d2f = jnp.where(colf == rows, BIGF, d2f)
            for t in range(K):
                mf = jnp.min(d2f, axis=1)
                idxf = jnp.min(jnp.where(d2f == mf[:, None], colf, BIGI),
                               axis=1)
                o_ref[pl.ds(s * _SUBR, _SUBR), pl.ds(t, 1)] = idxf[:, None]
                d2f = jnp.where(colf == idxf[:, None], BIGF, d2f)

        return carry

    lax.fori_loop(0, _KNN_R // _SUBR, sub_body, 0)


def _knn(qb, cb, sqq, sqc8):
    return pl.pallas_call(
        _knn_body,
        grid=(NP // _KNN_R,),
        in_specs=[
            pl.BlockSpec((_KNN_R, 8), lambda i: (i, 0)),
            pl.BlockSpec((8, NP), lambda i: (0, 0)),
            pl.BlockSpec((_KNN_R, 1), lambda i: (i, 0)),
            pl.BlockSpec((8, NP), lambda i: (0, 0)),
        ],
        out_specs=pl.BlockSpec((_KNN_R, KPAD), lambda i: (i, 0)),
        out_shape=jax.ShapeDtypeStruct((NP, KPAD), jnp.int32),
    )(qb, cb, sqq, sqc8)


_NC, _NS = 2, 16
_NW = _NC * _NS
_QPW = NP // _NW
_QS = 80
_NSUB = _QPW // _QS


def _agg_body(nbr_hbm, z_hbm, out_hbm, gidx, sidx, buf, acc_sh, sem):
    cid = lax.axis_index("c")
    sid = lax.axis_index("s")
    wid = cid * _NS + sid
    base = wid * _QPW
    sbase = sid * _QPW

    for j in range(K):
        pltpu.sync_copy(nbr_hbm.at[pl.ds(j * NP + base, _QPW)], gidx.at[j])

    for u in range(_NSUB):
        for t in range(_QS // 16):
            sidx[u, pl.ds(t * 16, 16)] = (
                lax.iota(jnp.int32, 16) + (sbase + u * _QS + t * 16))

    for u in range(_NSUB):
        pltpu.async_copy(z_hbm.at[gidx.at[0, pl.ds(u * _QS, _QS)]], buf, sem).wait()
        pltpu.sync_copy(buf, acc_sh.at[pl.ds(sbase + u * _QS, _QS)])

    def j_step(j, carry):
        for u in range(_NSUB):
            pltpu.async_copy(z_hbm.at[gidx.at[j, pl.ds(u * _QS, _QS)]], buf, sem).wait()
            pltpu.sync_copy(buf, acc_sh.at[sidx.at[u]], add=True)
        return carry

    lax.fori_loop(1, K, j_step, 0)

    pltpu.sync_copy(acc_sh.at[pl.ds(sbase, _QPW)], out_hbm.at[pl.ds(base, _QPW)])


def _sc_agg(z, nbr, width):
    mesh = plsc.VectorSubcoreMesh(core_axis_name="c", subcore_axis_name="s")
    fn = pl.kernel(
        _agg_body,
        out_type=jax.ShapeDtypeStruct((NP, width), jnp.float32),
        mesh=mesh,
        scratch_types=[
            pltpu.VMEM((KPAD, _QPW), jnp.int32),
            pltpu.VMEM((_NSUB, _QS), jnp.int32),
            pltpu.VMEM((_QS, width), jnp.float32),
            pltpu.VMEM_SHARED((_NS * _QPW, width), jnp.float32),
            pltpu.SemaphoreType.DMA,
        ],
        compiler_params=pltpu.CompilerParams(use_tc_tiling_on_sc=False),
    )
    return fn(nbr, z)


_ROWS = 1000


def _layer_body(has_agg, x_ref, *refs):
    if has_agg:
        g_ref, a_ref, c_ref, w_ref, b_ref, z_ref, s_ref = refs
        x = x_ref[...] + g_ref[...]
    else:
        a_ref, c_ref, w_ref, b_ref, z_ref, s_ref = refs
        x = x_ref[...]
    x = a_ref[0:1, :] * x + c_ref[0:1, :]
    z = jnp.dot(x.astype(jnp.bfloat16), w_ref[...],
                preferred_element_type=jnp.float32)
    z = z + b_ref[0:1, :]
    z = jnp.where(z >= 0, z, jnp.float32(0.33) * z)
    z_ref[...] = z
    cout = z.shape[1]
    s1 = jnp.sum(z, axis=0)[None, :]
    s2 = jnp.sum(z * z, axis=0)[None, :]
    r8 = lax.broadcasted_iota(jnp.int32, (8, cout), 0)
    s8 = jnp.where(r8 == 0, s1, jnp.where(r8 == 1, s2, jnp.float32(0.0)))
    i = pl.program_id(0)

    @pl.when(i == 0)
    def _init():
        s_ref[...] = jnp.zeros((8, cout), jnp.float32)

    s_ref[...] += s8


def _mlp_layer(x, agg, a, c, W, b):
    cin, cout = W.shape
    a8 = jnp.broadcast_to(a[None, :], (8, cin))
    c8 = jnp.broadcast_to(c[None, :], (8, cin))
    b8 = jnp.broadcast_to(b[None, :], (8, cout))
    wb = W.astype(jnp.bfloat16)
    ins = [x] + ([agg] if agg is not None else []) + [a8, c8, wb, b8]
    in_specs = [pl.BlockSpec((_ROWS, cin), lambda i: (i, 0))]
    if agg is not None:
        in_specs.append(pl.BlockSpec((_ROWS, cin), lambda i: (i, 0)))
    in_specs += [
        pl.BlockSpec((8, cin), lambda i: (0, 0)),
        pl.BlockSpec((8, cin), lambda i: (0, 0)),
        pl.BlockSpec((cin, cout), lambda i: (0, 0)),
        pl.BlockSpec((8, cout), lambda i: (0, 0)),
    ]
    return pl.pallas_call(
        functools.partial(_layer_body, agg is not None),
        grid=(N // _ROWS,),
        in_specs=in_specs,
        out_specs=[
            pl.BlockSpec((_ROWS, cout), lambda i: (i, 0)),
            pl.BlockSpec((8, cout), lambda i: (0, 0)),
        ],
        out_shape=[
            jax.ShapeDtypeStruct((N, cout), jnp.float32),
            jax.ShapeDtypeStruct((8, cout), jnp.float32),
        ],
    )(*ins)


def _final_body(x_ref, a_ref, c_ref, w_ref, b_ref, o_ref):
    x = a_ref[0:1, :] * x_ref[...] + c_ref[0:1, :]
    z = jnp.dot(x.astype(jnp.bfloat16), w_ref[...],
                preferred_element_type=jnp.float32)
    o_ref[...] = z + b_ref[0:1, :]


def _final_layer(x, a, c, W, b):
    cin, cout = W.shape
    a8 = jnp.broadcast_to(a[None, :], (8, cin))
    c8 = jnp.broadcast_to(c[None, :], (8, cin))
    b8 = jnp.broadcast_to(b[None, :], (8, cout))
    return pl.pallas_call(
        _final_body,
        grid=(N // _ROWS,),
        in_specs=[
            pl.BlockSpec((_ROWS, cin), lambda i: (i, 0)),
            pl.BlockSpec((8, cin), lambda i: (0, 0)),
            pl.BlockSpec((8, cin), lambda i: (0, 0)),
            pl.BlockSpec((cin, cout), lambda i: (0, 0)),
            pl.BlockSpec((8, cout), lambda i: (0, 0)),
        ],
        out_specs=pl.BlockSpec((_ROWS, cout), lambda i: (i, 0)),
        out_shape=jax.ShapeDtypeStruct((N, cout), jnp.float32),
    )(x, a8, c8, W.astype(jnp.bfloat16), b8)


def _stats_to_affine(sums, g, be):
    m = sums[0] / N
    v = sums[1] / N - m * m
    a = g / jnp.sqrt(v + 1e-5)
    return a, be - m * a


def kernel(input, params):
    pc = input
    coords = pc[:, 0:3]
    sq = jnp.sum(coords * coords, axis=1)

    cpad = jnp.pad(coords, ((0, NP - N), (0, 0)))
    qb = jnp.pad(cpad, ((0, 0), (0, 5))).astype(jnp.bfloat16)
    cb = jnp.pad(cpad.T, ((0, 5), (0, 0))).astype(jnp.bfloat16)
    sqq = jnp.pad(sq, (0, NP - N))[:, None]
    sqc8 = jnp.broadcast_to(
        jnp.pad(sq, (0, NP - N), constant_values=1e30)[None, :], (8, NP))

    nbr = _knn(qb, cb, sqq, sqc8)
    nbr_flat = nbr.T.reshape(-1)

    nc = (coords - 384.0) / 384.0
    x0 = jnp.concatenate([nc, pc[:, 4:5], jnp.zeros((N, 12), jnp.float32)], axis=1)

    z = x0
    a = jnp.ones((16,), jnp.float32)
    c = jnp.zeros((16,), jnp.float32)

    for name in ("gin1", "gin2", "gin3"):
        layers = params[name]
        aggz = _sc_agg(z, nbr_flat, z.shape[1])[:N]
        for li, (W, b, g, be) in enumerate(layers):
            if name == "gin1" and li == 0:
                W = jnp.pad(W, ((0, 12), (0, 0)))
            if li == 0:
                z, sums = _mlp_layer(z, aggz, a, (1.0 + K) * c, W, b)
            else:
                z, sums = _mlp_layer(z, None, a, c, W, b)
            a, c = _stats_to_affine(sums, g, be)

    for (W, b, g, be) in params["mlp3"]:
        z, sums = _mlp_layer(z, None, a, c, W, b)
        a, c = _stats_to_affine(sums, g, be)

    Wfin, bfin = params["final"]
    return _final_layer(z, a, c, Wfin, bfin)


# speedup vs baseline: 1.0314x; 1.0314x over previous
import functools

import jax
import jax.numpy as jnp
from jax import lax
from jax.experimental import pallas as pl
from jax.experimental.pallas import tpu as pltpu
from jax.experimental.pallas import tpu_sc as plsc

N = 10000
NP = 10240
K = 10
KPAD = 16


_KNN_R = 256


_SUBR = 16
_NG = NP // 128


def _knn_body(q_ref, c_ref, sqq_ref, sqc_ref, o_ref, d2_ref):
    pid = pl.program_id(0)
    lane = lax.broadcasted_iota(jnp.int32, (_SUBR, 128), 1)
    BIGF = jnp.float32(4e30)
    BIGI = jnp.int32(NP)

    dot = jnp.dot(q_ref[...], c_ref[...], preferred_element_type=jnp.float32)
    colsf = lax.broadcasted_iota(jnp.int32, (_KNN_R, NP), 1)
    rowsf = lax.broadcasted_iota(jnp.int32, (_KNN_R, NP), 0) + pid * _KNN_R
    d2all = (sqq_ref[...] + sqc_ref[0:1, :]) - 2.0 * dot
    d2_ref[...] = jnp.where(colsf == rowsf, BIGF, d2all)

    def sub_body(s, carry):
        V1 = V2 = V3 = V4 = jnp.full((_SUBR, 128), BIGF, jnp.float32)
        I1 = I2 = I3 = I4 = jnp.full((_SUBR, 128), BIGI, jnp.int32)
        for g in range(_NG):
            d2 = d2_ref[pl.ds(s * _SUBR, _SUBR), g * 128:(g + 1) * 128]
            colg = lane + (g * 128)
            b1 = d2 < V1
            b2 = d2 < V2
            b3 = d2 < V3
            b4 = d2 < V4
            V4 = jnp.where(b4, jnp.where(b3, V3, d2), V4)
            I4 = jnp.where(b4, jnp.where(b3, I3, colg), I4)
            V3 = jnp.where(b3, jnp.where(b2, V2, d2), V3)
            I3 = jnp.where(b3, jnp.where(b2, I2, colg), I3)
            V2 = jnp.where(b2, jnp.where(b1, V1, d2), V2)
            I2 = jnp.where(b2, jnp.where(b1, I1, colg), I2)
            V1 = jnp.where(b1, d2, V1)
            I1 = jnp.where(b1, colg, I1)
        VV = jnp.concatenate([V1, V2, V3, V4], axis=1)
        II = jnp.concatenate([I1, I2, I3, I4], axis=1)
        m = idx = None
        for t in range(K):
            m = jnp.min(VV, axis=1)
            idx = jnp.min(jnp.where(VV == m[:, None], II, BIGI), axis=1)
            o_ref[pl.ds(s * _SUBR, _SUBR), pl.ds(t, 1)] = idx[:, None]
            VV = jnp.where((VV == m[:, None]) & (II == idx[:, None]), BIGF, VV)
        bad = (V4 < m[:, None]) | ((V4 == m[:, None]) & (I4 <= idx[:, None]))
        flag = jnp.any(bad)

        @pl.when(flag)
        def _fallback():
            d2f = d2_ref[pl.ds(s * _SUBR, _SUBR), :]
            colf = lax.broadcasted_iota(jnp.int32, (_SUBR, NP), 1)
            for t in range(K):
                mf = jnp.min(d2f, axis=1)
                idxf = jnp.min(jnp.where(d2f == mf[:, None], colf, BIGI),
                               axis=1)
                o_ref[pl.ds(s * _SUBR, _SUBR), pl.ds(t, 1)] = idxf[:, None]
                d2f = jnp.where(colf == idxf[:, None], BIGF, d2f)

        return carry

    lax.fori_loop(0, _KNN_R // _SUBR, sub_body, 0)


def _knn(qb, cb, sqq, sqc8):
    return pl.pallas_call(
        _knn_body,
        grid=(NP // _KNN_R,),
        in_specs=[
            pl.BlockSpec((_KNN_R, 8), lambda i: (i, 0)),
            pl.BlockSpec((8, NP), lambda i: (0, 0)),
            pl.BlockSpec((_KNN_R, 1), lambda i: (i, 0)),
            pl.BlockSpec((8, NP), lambda i: (0, 0)),
        ],
        out_specs=pl.BlockSpec((_KNN_R, KPAD), lambda i: (i, 0)),
        out_shape=jax.ShapeDtypeStruct((NP, KPAD), jnp.int32),
        scratch_shapes=[pltpu.VMEM((_KNN_R, NP), jnp.float32)],
    )(qb, cb, sqq, sqc8)


_NC, _NS = 2, 16
_NW = _NC * _NS
_QPW = NP // _NW
_QS = 80
_NSUB = _QPW // _QS


def _agg_body(nbr_hbm, z_hbm, out_hbm, gidx, sidx, buf, acc_sh, sem):
    cid = lax.axis_index("c")
    sid = lax.axis_index("s")
    wid = cid * _NS + sid
    base = wid * _QPW
    sbase = sid * _QPW

    for j in range(K):
        pltpu.sync_copy(nbr_hbm.at[pl.ds(j * NP + base, _QPW)], gidx.at[j])

    for u in range(_NSUB):
        for t in range(_QS // 16):
            sidx[u, pl.ds(t * 16, 16)] = (
                lax.iota(jnp.int32, 16) + (sbase + u * _QS + t * 16))

    for u in range(_NSUB):
        pltpu.async_copy(z_hbm.at[gidx.at[0, pl.ds(u * _QS, _QS)]], buf, sem).wait()
        pltpu.sync_copy(buf, acc_sh.at[pl.ds(sbase + u * _QS, _QS)])

    def j_step(j, carry):
        for u in range(_NSUB):
            pltpu.async_copy(z_hbm.at[gidx.at[j, pl.ds(u * _QS, _QS)]], buf, sem).wait()
            pltpu.sync_copy(buf, acc_sh.at[sidx.at[u]], add=True)
        return carry

    lax.fori_loop(1, K, j_step, 0)

    pltpu.sync_copy(acc_sh.at[pl.ds(sbase, _QPW)], out_hbm.at[pl.ds(base, _QPW)])


def _sc_agg(z, nbr, width):
    mesh = plsc.VectorSubcoreMesh(core_axis_name="c", subcore_axis_name="s")
    fn = pl.kernel(
        _agg_body,
        out_type=jax.ShapeDtypeStruct((NP, width), jnp.float32),
        mesh=mesh,
        scratch_types=[
            pltpu.VMEM((KPAD, _QPW), jnp.int32),
            pltpu.VMEM((_NSUB, _QS), jnp.int32),
            pltpu.VMEM((_QS, width), jnp.float32),
            pltpu.VMEM_SHARED((_NS * _QPW, width), jnp.float32),
            pltpu.SemaphoreType.DMA,
        ],
        compiler_params=pltpu.CompilerParams(use_tc_tiling_on_sc=False),
    )
    return fn(nbr, z)


_ROWS = 1000


def _layer_body(has_agg, x_ref, *refs):
    if has_agg:
        g_ref, a_ref, c_ref, w_ref, b_ref, z_ref, s_ref = refs
        x = x_ref[...] + g_ref[...]
    else:
        a_ref, c_ref, w_ref, b_ref, z_ref, s_ref = refs
        x = x_ref[...]
    x = a_ref[0:1, :] * x + c_ref[0:1, :]
    z = jnp.dot(x.astype(jnp.bfloat16), w_ref[...],
                preferred_element_type=jnp.float32)
    z = z + b_ref[0:1, :]
    z = jnp.where(z >= 0, z, jnp.float32(0.33) * z)
    z_ref[...] = z
    cout = z.shape[1]
    s1 = jnp.sum(z, axis=0)[None, :]
    s2 = jnp.sum(z * z, axis=0)[None, :]
    r8 = lax.broadcasted_iota(jnp.int32, (8, cout), 0)
    s8 = jnp.where(r8 == 0, s1, jnp.where(r8 == 1, s2, jnp.float32(0.0)))
    i = pl.program_id(0)

    @pl.when(i == 0)
    def _init():
        s_ref[...] = jnp.zeros((8, cout), jnp.float32)

    s_ref[...] += s8


def _mlp_layer(x, agg, a, c, W, b):
    cin, cout = W.shape
    a8 = jnp.broadcast_to(a[None, :], (8, cin))
    c8 = jnp.broadcast_to(c[None, :], (8, cin))
    b8 = jnp.broadcast_to(b[None, :], (8, cout))
    wb = W.astype(jnp.bfloat16)
    ins = [x] + ([agg] if agg is not None else []) + [a8, c8, wb, b8]
    in_specs = [pl.BlockSpec((_ROWS, cin), lambda i: (i, 0))]
    if agg is not None:
        in_specs.append(pl.BlockSpec((_ROWS, cin), lambda i: (i, 0)))
    in_specs += [
        pl.BlockSpec((8, cin), lambda i: (0, 0)),
        pl.BlockSpec((8, cin), lambda i: (0, 0)),
        pl.BlockSpec((cin, cout), lambda i: (0, 0)),
        pl.BlockSpec((8, cout), lambda i: (0, 0)),
    ]
    return pl.pallas_call(
        functools.partial(_layer_body, agg is not None),
        grid=(N // _ROWS,),
        in_specs=in_specs,
        out_specs=[
            pl.BlockSpec((_ROWS, cout), lambda i: (i, 0)),
            pl.BlockSpec((8, cout), lambda i: (0, 0)),
        ],
        out_shape=[
            jax.ShapeDtypeStruct((N, cout), jnp.float32),
            jax.ShapeDtypeStruct((8, cout), jnp.float32),
        ],
    )(*ins)


def _final_body(x_ref, a_ref, c_ref, w_ref, b_ref, o_ref):
    x = a_ref[0:1, :] * x_ref[...] + c_ref[0:1, :]
    z = jnp.dot(x.astype(jnp.bfloat16), w_ref[...],
                preferred_element_type=jnp.float32)
    o_ref[...] = z + b_ref[0:1, :]


def _final_layer(x, a, c, W, b):
    cin, cout = W.shape
    a8 = jnp.broadcast_to(a[None, :], (8, cin))
    c8 = jnp.broadcast_to(c[None, :], (8, cin))
    b8 = jnp.broadcast_to(b[None, :], (8, cout))
    return pl.pallas_call(
        _final_body,
        grid=(N // _ROWS,),
        in_specs=[
            pl.BlockSpec((_ROWS, cin), lambda i: (i, 0)),
            pl.BlockSpec((8, cin), lambda i: (0, 0)),
            pl.BlockSpec((8, cin), lambda i: (0, 0)),
            pl.BlockSpec((cin, cout), lambda i: (0, 0)),
            pl.BlockSpec((8, cout), lambda i: (0, 0)),
        ],
        out_specs=pl.BlockSpec((_ROWS, cout), lambda i: (i, 0)),
        out_shape=jax.ShapeDtypeStruct((N, cout), jnp.float32),
    )(x, a8, c8, W.astype(jnp.bfloat16), b8)


def _stats_to_affine(sums, g, be):
    m = sums[0] / N
    v = sums[1] / N - m * m
    a = g / jnp.sqrt(v + 1e-5)
    return a, be - m * a


def kernel(input, params):
    pc = input
    coords = pc[:, 0:3]
    sq = jnp.sum(coords * coords, axis=1)

    cpad = jnp.pad(coords, ((0, NP - N), (0, 0)))
    qb = jnp.pad(cpad, ((0, 0), (0, 5))).astype(jnp.bfloat16)
    cb = jnp.pad(cpad.T, ((0, 5), (0, 0))).astype(jnp.bfloat16)
    sqq = jnp.pad(sq, (0, NP - N))[:, None]
    sqc8 = jnp.broadcast_to(
        jnp.pad(sq, (0, NP - N), constant_values=1e30)[None, :], (8, NP))

    nbr = _knn(qb, cb, sqq, sqc8)
    nbr_flat = nbr.T.reshape(-1)

    nc = (coords - 384.0) / 384.0
    x0 = jnp.concatenate([nc, pc[:, 4:5], jnp.zeros((N, 12), jnp.float32)], axis=1)

    z = x0
    a = jnp.ones((16,), jnp.float32)
    c = jnp.zeros((16,), jnp.float32)

    for name in ("gin1", "gin2", "gin3"):
        layers = params[name]
        aggz = _sc_agg(z, nbr_flat, z.shape[1])[:N]
        for li, (W, b, g, be) in enumerate(layers):
            if name == "gin1" and li == 0:
                W = jnp.pad(W, ((0, 12), (0, 0)))
            if li == 0:
                z, sums = _mlp_layer(z, aggz, a, (1.0 + K) * c, W, b)
            else:
                z, sums = _mlp_layer(z, None, a, c, W, b)
            a, c = _stats_to_affine(sums, g, be)

    for (W, b, g, be) in params["mlp3"]:
        z, sums = _mlp_layer(z, None, a, c, W, b)
        a, c = _stats_to_affine(sums, g, be)

    Wfin, bfin = params["final"]
    return _final_layer(z, a, c, Wfin, bfin)


# speedup vs baseline: 1.5440x; 1.4970x over previous
import functools

import jax
import jax.numpy as jnp
from jax import lax
from jax.experimental import pallas as pl
from jax.experimental.pallas import tpu as pltpu
from jax.experimental.pallas import tpu_sc as plsc

N = 10000
NP = 10240
K = 10
KPAD = 16


_KNN_R = 256


_SUBR = 32
_NG = NP // 128


def _knn_body(q_ref, c_ref, sqq_ref, sqc_ref, o_ref, d2_ref):
    pid = pl.program_id(0)
    lane = lax.broadcasted_iota(jnp.int32, (_SUBR, 128), 1)
    BIGF = jnp.float32(4e30)
    BIGI = jnp.int32(NP)

    dot = jnp.dot(q_ref[...], c_ref[...], preferred_element_type=jnp.float32)
    colsf = lax.broadcasted_iota(jnp.int32, (_KNN_R, NP), 1)
    rowsf = lax.broadcasted_iota(jnp.int32, (_KNN_R, NP), 0) + pid * _KNN_R
    d2all = (sqq_ref[...] + sqc_ref[0:1, :]) - 2.0 * dot
    d2_ref[...] = jnp.where(colsf == rowsf, BIGF, d2all)

    def sub_body(s, carry):
        V1 = V2 = V3 = V4 = jnp.full((_SUBR, 128), BIGF, jnp.float32)
        I1 = I2 = I3 = I4 = jnp.full((_SUBR, 128), BIGI, jnp.int32)
        for g in range(_NG):
            d2 = d2_ref[pl.ds(s * _SUBR, _SUBR), g * 128:(g + 1) * 128]
            colg = lane + (g * 128)
            b1 = d2 < V1
            b2 = d2 < V2
            b3 = d2 < V3
            b4 = d2 < V4
            V4 = jnp.where(b4, jnp.where(b3, V3, d2), V4)
            I4 = jnp.where(b4, jnp.where(b3, I3, colg), I4)
            V3 = jnp.where(b3, jnp.where(b2, V2, d2), V3)
            I3 = jnp.where(b3, jnp.where(b2, I2, colg), I3)
            V2 = jnp.where(b2, jnp.where(b1, V1, d2), V2)
            I2 = jnp.where(b2, jnp.where(b1, I1, colg), I2)
            V1 = jnp.where(b1, d2, V1)
            I1 = jnp.where(b1, colg, I1)
        VV = jnp.concatenate([V1, V2, V3, V4], axis=1)
        II = jnp.concatenate([I1, I2, I3, I4], axis=1)
        m = idx = None
        for t in range(K):
            m = jnp.min(VV, axis=1)
            idx = jnp.min(jnp.where(VV == m[:, None], II, BIGI), axis=1)
            o_ref[pl.ds(s * _SUBR, _SUBR), pl.ds(t, 1)] = idx[:, None]
            VV = jnp.where((VV == m[:, None]) & (II == idx[:, None]), BIGF, VV)
        bad = (V4 < m[:, None]) | ((V4 == m[:, None]) & (I4 <= idx[:, None]))
        flag = jnp.any(bad)

        @pl.when(flag)
        def _fallback():
            d2f = d2_ref[pl.ds(s * _SUBR, _SUBR), :]
            colf = lax.broadcasted_iota(jnp.int32, (_SUBR, NP), 1)
            for t in range(K):
                mf = jnp.min(d2f, axis=1)
                idxf = jnp.min(jnp.where(d2f == mf[:, None], colf, BIGI),
                               axis=1)
                o_ref[pl.ds(s * _SUBR, _SUBR), pl.ds(t, 1)] = idxf[:, None]
                d2f = jnp.where(colf == idxf[:, None], BIGF, d2f)

        return carry

    lax.fori_loop(0, _KNN_R // _SUBR, sub_body, 0)


def _knn(qb, cb, sqq, sqc8):
    return pl.pallas_call(
        _knn_body,
        grid=(NP // _KNN_R,),
        in_specs=[
            pl.BlockSpec((_KNN_R, 8), lambda i: (i, 0)),
            pl.BlockSpec((8, NP), lambda i: (0, 0)),
            pl.BlockSpec((_KNN_R, 1), lambda i: (i, 0)),
            pl.BlockSpec((8, NP), lambda i: (0, 0)),
        ],
        out_specs=pl.BlockSpec((_KNN_R, KPAD), lambda i: (i, 0)),
        out_shape=jax.ShapeDtypeStruct((NP, KPAD), jnp.int32),
        scratch_shapes=[pltpu.VMEM((_KNN_R, NP), jnp.float32)],
    )(qb, cb, sqq, sqc8)


_NC, _NS = 2, 16
_NW = _NC * _NS
_QPW = NP // _NW
_QS = 80
_NSUB = _QPW // _QS


def _agg_body(nbr_hbm, z_hbm, out_hbm, gidx, sidx, buf, acc_sh, sem):
    cid = lax.axis_index("c")
    sid = lax.axis_index("s")
    wid = cid * _NS + sid
    base = wid * _QPW
    sbase = sid * _QPW

    for j in range(K):
        pltpu.sync_copy(nbr_hbm.at[pl.ds(j * NP + base, _QPW)], gidx.at[j])

    for u in range(_NSUB):
        for t in range(_QS // 16):
            sidx[u, pl.ds(t * 16, 16)] = (
                lax.iota(jnp.int32, 16) + (sbase + u * _QS + t * 16))

    for u in range(_NSUB):
        pltpu.async_copy(z_hbm.at[gidx.at[0, pl.ds(u * _QS, _QS)]], buf, sem).wait()
        pltpu.sync_copy(buf, acc_sh.at[pl.ds(sbase + u * _QS, _QS)])

    def j_step(j, carry):
        for u in range(_NSUB):
            pltpu.async_copy(z_hbm.at[gidx.at[j, pl.ds(u * _QS, _QS)]], buf, sem).wait()
            pltpu.sync_copy(buf, acc_sh.at[sidx.at[u]], add=True)
        return carry

    lax.fori_loop(1, K, j_step, 0)

    pltpu.sync_copy(acc_sh.at[pl.ds(sbase, _QPW)], out_hbm.at[pl.ds(base, _QPW)])


def _sc_agg(z, nbr, width):
    mesh = plsc.VectorSubcoreMesh(core_axis_name="c", subcore_axis_name="s")
    fn = pl.kernel(
        _agg_body,
        out_type=jax.ShapeDtypeStruct((NP, width), jnp.float32),
        mesh=mesh,
        scratch_types=[
            pltpu.VMEM((KPAD, _QPW), jnp.int32),
            pltpu.VMEM((_NSUB, _QS), jnp.int32),
            pltpu.VMEM((_QS, width), jnp.float32),
            pltpu.VMEM_SHARED((_NS * _QPW, width), jnp.float32),
            pltpu.SemaphoreType.DMA,
        ],
        compiler_params=pltpu.CompilerParams(use_tc_tiling_on_sc=False),
    )
    return fn(nbr, z)


_ROWS = 1000


def _layer_body(has_agg, x_ref, *refs):
    if has_agg:
        g_ref, a_ref, c_ref, w_ref, b_ref, z_ref, s_ref = refs
        x = x_ref[...] + g_ref[...]
    else:
        a_ref, c_ref, w_ref, b_ref, z_ref, s_ref = refs
        x = x_ref[...]
    x = a_ref[0:1, :] * x + c_ref[0:1, :]
    z = jnp.dot(x.astype(jnp.bfloat16), w_ref[...],
                preferred_element_type=jnp.float32)
    z = z + b_ref[0:1, :]
    z = jnp.where(z >= 0, z, jnp.float32(0.33) * z)
    z_ref[...] = z
    cout = z.shape[1]
    s1 = jnp.sum(z, axis=0)[None, :]
    s2 = jnp.sum(z * z, axis=0)[None, :]
    r8 = lax.broadcasted_iota(jnp.int32, (8, cout), 0)
    s8 = jnp.where(r8 == 0, s1, jnp.where(r8 == 1, s2, jnp.float32(0.0)))
    i = pl.program_id(0)

    @pl.when(i == 0)
    def _init():
        s_ref[...] = jnp.zeros((8, cout), jnp.float32)

    s_ref[...] += s8


def _mlp_layer(x, agg, a, c, W, b):
    cin, cout = W.shape
    a8 = jnp.broadcast_to(a[None, :], (8, cin))
    c8 = jnp.broadcast_to(c[None, :], (8, cin))
    b8 = jnp.broadcast_to(b[None, :], (8, cout))
    wb = W.astype(jnp.bfloat16)
    ins = [x] + ([agg] if agg is not None else []) + [a8, c8, wb, b8]
    in_specs = [pl.BlockSpec((_ROWS, cin), lambda i: (i, 0))]
    if agg is not None:
        in_specs.append(pl.BlockSpec((_ROWS, cin), lambda i: (i, 0)))
    in_specs += [
        pl.BlockSpec((8, cin), lambda i: (0, 0)),
        pl.BlockSpec((8, cin), lambda i: (0, 0)),
        pl.BlockSpec((cin, cout), lambda i: (0, 0)),
        pl.BlockSpec((8, cout), lambda i: (0, 0)),
    ]
    return pl.pallas_call(
        functools.partial(_layer_body, agg is not None),
        grid=(N // _ROWS,),
        in_specs=in_specs,
        out_specs=[
            pl.BlockSpec((_ROWS, cout), lambda i: (i, 0)),
            pl.BlockSpec((8, cout), lambda i: (0, 0)),
        ],
        out_shape=[
            jax.ShapeDtypeStruct((N, cout), jnp.float32),
            jax.ShapeDtypeStruct((8, cout), jnp.float32),
        ],
    )(*ins)


def _final_body(x_ref, a_ref, c_ref, w_ref, b_ref, o_ref):
    x = a_ref[0:1, :] * x_ref[...] + c_ref[0:1, :]
    z = jnp.dot(x.astype(jnp.bfloat16), w_ref[...],
                preferred_element_type=jnp.float32)
    o_ref[...] = z + b_ref[0:1, :]


def _final_layer(x, a, c, W, b):
    cin, cout = W.shape
    a8 = jnp.broadcast_to(a[None, :], (8, cin))
    c8 = jnp.broadcast_to(c[None, :], (8, cin))
    b8 = jnp.broadcast_to(b[None, :], (8, cout))
    return pl.pallas_call(
        _final_body,
        grid=(N // _ROWS,),
        in_specs=[
            pl.BlockSpec((_ROWS, cin), lambda i: (i, 0)),
            pl.BlockSpec((8, cin), lambda i: (0, 0)),
            pl.BlockSpec((8, cin), lambda i: (0, 0)),
            pl.BlockSpec((cin, cout), lambda i: (0, 0)),
            pl.BlockSpec((8, cout), lambda i: (0, 0)),
        ],
        out_specs=pl.BlockSpec((_ROWS, cout), lambda i: (i, 0)),
        out_shape=jax.ShapeDtypeStruct((N, cout), jnp.float32),
    )(x, a8, c8, W.astype(jnp.bfloat16), b8)


def _stats_to_affine(sums, g, be):
    m = sums[0] / N
    v = sums[1] / N - m * m
    a = g / jnp.sqrt(v + 1e-5)
    return a, be - m * a


def kernel(input, params):
    pc = input
    coords = pc[:, 0:3]
    sq = jnp.sum(coords * coords, axis=1)

    cpad = jnp.pad(coords, ((0, NP - N), (0, 0)))
    qb = jnp.pad(cpad, ((0, 0), (0, 5))).astype(jnp.bfloat16)
    cb = jnp.pad(cpad.T, ((0, 5), (0, 0))).astype(jnp.bfloat16)
    sqq = jnp.pad(sq, (0, NP - N))[:, None]
    sqc8 = jnp.broadcast_to(
        jnp.pad(sq, (0, NP - N), constant_values=1e30)[None, :], (8, NP))

    nbr = _knn(qb, cb, sqq, sqc8)
    nbr_flat = nbr.T.reshape(-1)

    nc = (coords - 384.0) / 384.0
    x0 = jnp.concatenate([nc, pc[:, 4:5], jnp.zeros((N, 12), jnp.float32)], axis=1)

    z = x0
    a = jnp.ones((16,), jnp.float32)
    c = jnp.zeros((16,), jnp.float32)

    for name in ("gin1", "gin2", "gin3"):
        layers = params[name]
        aggz = _sc_agg(z, nbr_flat, z.shape[1])[:N]
        for li, (W, b, g, be) in enumerate(layers):
            if name == "gin1" and li == 0:
                W = jnp.pad(W, ((0, 12), (0, 0)))
            if li == 0:
                z, sums = _mlp_layer(z, aggz, a, (1.0 + K) * c, W, b)
            else:
                z, sums = _mlp_layer(z, None, a, c, W, b)
            a, c = _stats_to_affine(sums, g, be)

    for (W, b, g, be) in params["mlp3"]:
        z, sums = _mlp_layer(z, None, a, c, W, b)
        a, c = _stats_to_affine(sums, g, be)

    Wfin, bfin = params["final"]
    return _final_layer(z, a, c, Wfin, bfin)


# speedup vs baseline: 2.0552x; 1.3311x over previous
import functools

import jax
import jax.numpy as jnp
from jax import lax
from jax.experimental import pallas as pl
from jax.experimental.pallas import tpu as pltpu
from jax.experimental.pallas import tpu_sc as plsc

N = 10000
NP = 10240
K = 10
KPAD = 16


_KNN_R = 256


_SUBR = 64
_NG = NP // 128


def _knn_body(q_ref, c_ref, sqq_ref, sqc_ref, o_ref, d2_ref):
    pid = pl.program_id(0)
    lane = lax.broadcasted_iota(jnp.int32, (_SUBR, 128), 1)
    BIGF = jnp.float32(4e30)
    BIGI = jnp.int32(NP)

    dot = jnp.dot(q_ref[...], c_ref[...], preferred_element_type=jnp.float32)
    colsf = lax.broadcasted_iota(jnp.int32, (_KNN_R, NP), 1)
    rowsf = lax.broadcasted_iota(jnp.int32, (_KNN_R, NP), 0) + pid * _KNN_R
    d2all = (sqq_ref[...] + sqc_ref[0:1, :]) - 2.0 * dot
    d2_ref[...] = jnp.where(colsf == rowsf, BIGF, d2all)

    def sub_body(s, carry):
        V1 = V2 = V3 = V4 = jnp.full((_SUBR, 128), BIGF, jnp.float32)
        I1 = I2 = I3 = I4 = jnp.full((_SUBR, 128), BIGI, jnp.int32)
        for g in range(_NG):
            d2 = d2_ref[pl.ds(s * _SUBR, _SUBR), g * 128:(g + 1) * 128]
            colg = lane + (g * 128)
            b1 = d2 < V1
            b2 = d2 < V2
            b3 = d2 < V3
            b4 = d2 < V4
            V4 = jnp.where(b4, jnp.where(b3, V3, d2), V4)
            I4 = jnp.where(b4, jnp.where(b3, I3, colg), I4)
            V3 = jnp.where(b3, jnp.where(b2, V2, d2), V3)
            I3 = jnp.where(b3, jnp.where(b2, I2, colg), I3)
            V2 = jnp.where(b2, jnp.where(b1, V1, d2), V2)
            I2 = jnp.where(b2, jnp.where(b1, I1, colg), I2)
            V1 = jnp.where(b1, d2, V1)
            I1 = jnp.where(b1, colg, I1)
        VV = jnp.concatenate([V1, V2, V3, V4], axis=1)
        II = jnp.concatenate([I1, I2, I3, I4], axis=1)
        m = idx = None
        for t in range(K):
            m = jnp.min(VV, axis=1)
            idx = jnp.min(jnp.where(VV == m[:, None], II, BIGI), axis=1)
            o_ref[pl.ds(s * _SUBR, _SUBR), pl.ds(t, 1)] = idx[:, None]
            VV = jnp.where((VV == m[:, None]) & (II == idx[:, None]), BIGF, VV)
        bad = (V4 < m[:, None]) | ((V4 == m[:, None]) & (I4 <= idx[:, None]))
        flag = jnp.any(bad)

        @pl.when(flag)
        def _fallback():
            d2f = d2_ref[pl.ds(s * _SUBR, _SUBR), :]
            colf = lax.broadcasted_iota(jnp.int32, (_SUBR, NP), 1)
            for t in range(K):
                mf = jnp.min(d2f, axis=1)
                idxf = jnp.min(jnp.where(d2f == mf[:, None], colf, BIGI),
                               axis=1)
                o_ref[pl.ds(s * _SUBR, _SUBR), pl.ds(t, 1)] = idxf[:, None]
                d2f = jnp.where(colf == idxf[:, None], BIGF, d2f)

        return carry

    lax.fori_loop(0, _KNN_R // _SUBR, sub_body, 0)


def _knn(qb, cb, sqq, sqc8):
    return pl.pallas_call(
        _knn_body,
        grid=(NP // _KNN_R,),
        in_specs=[
            pl.BlockSpec((_KNN_R, 8), lambda i: (i, 0)),
            pl.BlockSpec((8, NP), lambda i: (0, 0)),
            pl.BlockSpec((_KNN_R, 1), lambda i: (i, 0)),
            pl.BlockSpec((8, NP), lambda i: (0, 0)),
        ],
        out_specs=pl.BlockSpec((_KNN_R, KPAD), lambda i: (i, 0)),
        out_shape=jax.ShapeDtypeStruct((NP, KPAD), jnp.int32),
        scratch_shapes=[pltpu.VMEM((_KNN_R, NP), jnp.float32)],
    )(qb, cb, sqq, sqc8)


_NC, _NS = 2, 16
_NW = _NC * _NS
_QPW = NP // _NW
_QS = 80
_NSUB = _QPW // _QS


def _agg_body(nbr_hbm, z_hbm, out_hbm, gidx, sidx, buf, acc_sh, sem):
    cid = lax.axis_index("c")
    sid = lax.axis_index("s")
    wid = cid * _NS + sid
    base = wid * _QPW
    sbase = sid * _QPW

    for j in range(K):
        pltpu.sync_copy(nbr_hbm.at[pl.ds(j * NP + base, _QPW)], gidx.at[j])

    for u in range(_NSUB):
        for t in range(_QS // 16):
            sidx[u, pl.ds(t * 16, 16)] = (
                lax.iota(jnp.int32, 16) + (sbase + u * _QS + t * 16))

    for u in range(_NSUB):
        pltpu.async_copy(z_hbm.at[gidx.at[0, pl.ds(u * _QS, _QS)]], buf, sem).wait()
        pltpu.sync_copy(buf, acc_sh.at[pl.ds(sbase + u * _QS, _QS)])

    def j_step(j, carry):
        for u in range(_NSUB):
            pltpu.async_copy(z_hbm.at[gidx.at[j, pl.ds(u * _QS, _QS)]], buf, sem).wait()
            pltpu.sync_copy(buf, acc_sh.at[sidx.at[u]], add=True)
        return carry

    lax.fori_loop(1, K, j_step, 0)

    pltpu.sync_copy(acc_sh.at[pl.ds(sbase, _QPW)], out_hbm.at[pl.ds(base, _QPW)])


def _sc_agg(z, nbr, width):
    mesh = plsc.VectorSubcoreMesh(core_axis_name="c", subcore_axis_name="s")
    fn = pl.kernel(
        _agg_body,
        out_type=jax.ShapeDtypeStruct((NP, width), jnp.float32),
        mesh=mesh,
        scratch_types=[
            pltpu.VMEM((KPAD, _QPW), jnp.int32),
            pltpu.VMEM((_NSUB, _QS), jnp.int32),
            pltpu.VMEM((_QS, width), jnp.float32),
            pltpu.VMEM_SHARED((_NS * _QPW, width), jnp.float32),
            pltpu.SemaphoreType.DMA,
        ],
        compiler_params=pltpu.CompilerParams(use_tc_tiling_on_sc=False),
    )
    return fn(nbr, z)


_ROWS = 1000


def _layer_body(has_agg, x_ref, *refs):
    if has_agg:
        g_ref, a_ref, c_ref, w_ref, b_ref, z_ref, s_ref = refs
        x = x_ref[...] + g_ref[...]
    else:
        a_ref, c_ref, w_ref, b_ref, z_ref, s_ref = refs
        x = x_ref[...]
    x = a_ref[0:1, :] * x + c_ref[0:1, :]
    z = jnp.dot(x.astype(jnp.bfloat16), w_ref[...],
                preferred_element_type=jnp.float32)
    z = z + b_ref[0:1, :]
    z = jnp.where(z >= 0, z, jnp.float32(0.33) * z)
    z_ref[...] = z
    cout = z.shape[1]
    s1 = jnp.sum(z, axis=0)[None, :]
    s2 = jnp.sum(z * z, axis=0)[None, :]
    r8 = lax.broadcasted_iota(jnp.int32, (8, cout), 0)
    s8 = jnp.where(r8 == 0, s1, jnp.where(r8 == 1, s2, jnp.float32(0.0)))
    i = pl.program_id(0)

    @pl.when(i == 0)
    def _init():
        s_ref[...] = jnp.zeros((8, cout), jnp.float32)

    s_ref[...] += s8


def _mlp_layer(x, agg, a, c, W, b):
    cin, cout = W.shape
    a8 = jnp.broadcast_to(a[None, :], (8, cin))
    c8 = jnp.broadcast_to(c[None, :], (8, cin))
    b8 = jnp.broadcast_to(b[None, :], (8, cout))
    wb = W.astype(jnp.bfloat16)
    ins = [x] + ([agg] if agg is not None else []) + [a8, c8, wb, b8]
    in_specs = [pl.BlockSpec((_ROWS, cin), lambda i: (i, 0))]
    if agg is not None:
        in_specs.append(pl.BlockSpec((_ROWS, cin), lambda i: (i, 0)))
    in_specs += [
        pl.BlockSpec((8, cin), lambda i: (0, 0)),
        pl.BlockSpec((8, cin), lambda i: (0, 0)),
        pl.BlockSpec((cin, cout), lambda i: (0, 0)),
        pl.BlockSpec((8, cout), lambda i: (0, 0)),
    ]
    return pl.pallas_call(
        functools.partial(_layer_body, agg is not None),
        grid=(N // _ROWS,),
        in_specs=in_specs,
        out_specs=[
            pl.BlockSpec((_ROWS, cout), lambda i: (i, 0)),
            pl.BlockSpec((8, cout), lambda i: (0, 0)),
        ],
        out_shape=[
            jax.ShapeDtypeStruct((N, cout), jnp.float32),
            jax.ShapeDtypeStruct((8, cout), jnp.float32),
        ],
    )(*ins)


def _final_body(x_ref, a_ref, c_ref, w_ref, b_ref, o_ref):
    x = a_ref[0:1, :] * x_ref[...] + c_ref[0:1, :]
    z = jnp.dot(x.astype(jnp.bfloat16), w_ref[...],
                preferred_element_type=jnp.float32)
    o_ref[...] = z + b_ref[0:1, :]


def _final_layer(x, a, c, W, b):
    cin, cout = W.shape
    a8 = jnp.broadcast_to(a[None, :], (8, cin))
    c8 = jnp.broadcast_to(c[None, :], (8, cin))
    b8 = jnp.broadcast_to(b[None, :], (8, cout))
    return pl.pallas_call(
        _final_body,
        grid=(N // _ROWS,),
        in_specs=[
            pl.BlockSpec((_ROWS, cin), lambda i: (i, 0)),
            pl.BlockSpec((8, cin), lambda i: (0, 0)),
            pl.BlockSpec((8, cin), lambda i: (0, 0)),
            pl.BlockSpec((cin, cout), lambda i: (0, 0)),
            pl.BlockSpec((8, cout), lambda i: (0, 0)),
        ],
        out_specs=pl.BlockSpec((_ROWS, cout), lambda i: (i, 0)),
        out_shape=jax.ShapeDtypeStruct((N, cout), jnp.float32),
    )(x, a8, c8, W.astype(jnp.bfloat16), b8)


def _stats_to_affine(sums, g, be):
    m = sums[0] / N
    v = sums[1] / N - m * m
    a = g / jnp.sqrt(v + 1e-5)
    return a, be - m * a


def kernel(input, params):
    pc = input
    coords = pc[:, 0:3]
    sq = jnp.sum(coords * coords, axis=1)

    cpad = jnp.pad(coords, ((0, NP - N), (0, 0)))
    qb = jnp.pad(cpad, ((0, 0), (0, 5))).astype(jnp.bfloat16)
    cb = jnp.pad(cpad.T, ((0, 5), (0, 0))).astype(jnp.bfloat16)
    sqq = jnp.pad(sq, (0, NP - N))[:, None]
    sqc8 = jnp.broadcast_to(
        jnp.pad(sq, (0, NP - N), constant_values=1e30)[None, :], (8, NP))

    nbr = _knn(qb, cb, sqq, sqc8)
    nbr_flat = nbr.T.reshape(-1)

    nc = (coords - 384.0) / 384.0
    x0 = jnp.concatenate([nc, pc[:, 4:5], jnp.zeros((N, 12), jnp.float32)], axis=1)

    z = x0
    a = jnp.ones((16,), jnp.float32)
    c = jnp.zeros((16,), jnp.float32)

    for name in ("gin1", "gin2", "gin3"):
        layers = params[name]
        aggz = _sc_agg(z, nbr_flat, z.shape[1])[:N]
        for li, (W, b, g, be) in enumerate(layers):
            if name == "gin1" and li == 0:
                W = jnp.pad(W, ((0, 12), (0, 0)))
            if li == 0:
                z, sums = _mlp_layer(z, aggz, a, (1.0 + K) * c, W, b)
            else:
                z, sums = _mlp_layer(z, None, a, c, W, b)
            a, c = _stats_to_affine(sums, g, be)

    for (W, b, g, be) in params["mlp3"]:
        z, sums = _mlp_layer(z, None, a, c, W, b)
        a, c = _stats_to_affine(sums, g, be)

    Wfin, bfin = params["final"]
    return _final_layer(z, a, c, Wfin, bfin)


# speedup vs baseline: 2.1413x; 1.0419x over previous
import functools

import jax
import jax.numpy as jnp
from jax import lax
from jax.experimental import pallas as pl
from jax.experimental.pallas import tpu as pltpu
from jax.experimental.pallas import tpu_sc as plsc

N = 10000
NP = 10240
K = 10
KPAD = 16


_KNN_R = 256


_SUBR = 64
_NG = NP // 128


def _knn_body(q_ref, c_ref, sqq_ref, sqc_ref, o_ref, d2_ref):
    pid = pl.program_id(0)
    lane = lax.broadcasted_iota(jnp.int32, (_SUBR, 128), 1)
    BIGF = jnp.float32(4e30)
    BIGI = jnp.int32(NP)

    dot = jnp.dot(q_ref[...], c_ref[...], preferred_element_type=jnp.float32)
    colsf = lax.broadcasted_iota(jnp.int32, (_KNN_R, NP), 1)
    rowsf = lax.broadcasted_iota(jnp.int32, (_KNN_R, NP), 0) + pid * _KNN_R
    d2all = (sqq_ref[...] + sqc_ref[0:1, :]) - 2.0 * dot
    d2_ref[...] = jnp.where(colsf == rowsf, BIGF, d2all)

    def sub_body(s, carry):
        V1 = V2 = V3 = V4 = jnp.full((_SUBR, 128), BIGF, jnp.float32)
        I1 = I2 = I3 = I4 = jnp.full((_SUBR, 128), BIGI, jnp.int32)
        for g in range(_NG):
            d2 = d2_ref[pl.ds(s * _SUBR, _SUBR), g * 128:(g + 1) * 128]
            colg = lane + (g * 128)
            b1 = d2 < V1
            b2 = d2 < V2
            b3 = d2 < V3
            b4 = d2 < V4
            V4 = jnp.where(b4, jnp.where(b3, V3, d2), V4)
            I4 = jnp.where(b4, jnp.where(b3, I3, colg), I4)
            V3 = jnp.where(b3, jnp.where(b2, V2, d2), V3)
            I3 = jnp.where(b3, jnp.where(b2, I2, colg), I3)
            V2 = jnp.where(b2, jnp.where(b1, V1, d2), V2)
            I2 = jnp.where(b2, jnp.where(b1, I1, colg), I2)
            V1 = jnp.where(b1, d2, V1)
            I1 = jnp.where(b1, colg, I1)
        VV = jnp.concatenate([V1, V2, V3, V4], axis=1)
        II = jnp.concatenate([I1, I2, I3, I4], axis=1)
        m = idx = None
        for t in range(K):
            m = jnp.min(VV, axis=1)
            idx = jnp.min(jnp.where(VV == m[:, None], II, BIGI), axis=1)
            o_ref[pl.ds(s * _SUBR, _SUBR), pl.ds(t, 1)] = idx[:, None]
            VV = jnp.where((VV == m[:, None]) & (II == idx[:, None]), BIGF, VV)
        bad = (V4 < m[:, None]) | ((V4 == m[:, None]) & (I4 <= idx[:, None]))
        flag = jnp.any(bad)

        @pl.when(flag)
        def _fallback():
            d2f = d2_ref[pl.ds(s * _SUBR, _SUBR), :]
            colf = lax.broadcasted_iota(jnp.int32, (_SUBR, NP), 1)
            for t in range(K):
                mf = jnp.min(d2f, axis=1)
                idxf = jnp.min(jnp.where(d2f == mf[:, None], colf, BIGI),
                               axis=1)
                o_ref[pl.ds(s * _SUBR, _SUBR), pl.ds(t, 1)] = idxf[:, None]
                d2f = jnp.where(colf == idxf[:, None], BIGF, d2f)

        return carry

    lax.fori_loop(0, _KNN_R // _SUBR, sub_body, 0)


def _knn(qb, cb, sqq, sqc8):
    return pl.pallas_call(
        _knn_body,
        grid=(NP // _KNN_R,),
        in_specs=[
            pl.BlockSpec((_KNN_R, 8), lambda i: (i, 0)),
            pl.BlockSpec((8, NP), lambda i: (0, 0)),
            pl.BlockSpec((_KNN_R, 1), lambda i: (i, 0)),
            pl.BlockSpec((8, NP), lambda i: (0, 0)),
        ],
        out_specs=pl.BlockSpec((_KNN_R, KPAD), lambda i: (i, 0)),
        out_shape=jax.ShapeDtypeStruct((NP, KPAD), jnp.int32),
        scratch_shapes=[pltpu.VMEM((_KNN_R, NP), jnp.float32)],
    )(qb, cb, sqq, sqc8)


_NC, _NS = 2, 16
_NW = _NC * _NS
_QPW = NP // _NW
_QS = 80
_NSUB = _QPW // _QS


def _agg_body(nbr_hbm, z_hbm, out_hbm, gidx, sidx, bufs, acc_sh,
              s0, s1, s2, s3):
    cid = lax.axis_index("c")
    sid = lax.axis_index("s")
    wid = cid * _NS + sid
    base = wid * _QPW
    sbase = sid * _QPW
    sems = (s0, s1, s2, s3)

    for j in range(K):
        pltpu.sync_copy(nbr_hbm.at[pl.ds(j * NP + base, _QPW)], gidx.at[j])

    for u in range(_NSUB):
        for t in range(_QS // 16):
            sidx[u, pl.ds(t * 16, 16)] = (
                lax.iota(jnp.int32, 16) + (sbase + u * _QS + t * 16))

    cps = [pltpu.async_copy(z_hbm.at[gidx.at[0, pl.ds(u * _QS, _QS)]],
                            bufs.at[u], sems[u]) for u in range(_NSUB)]
    for u in range(_NSUB):
        cps[u].wait()
        pltpu.sync_copy(bufs.at[u], acc_sh.at[pl.ds(sbase + u * _QS, _QS)])

    def j_step(j, carry):
        cps = [pltpu.async_copy(z_hbm.at[gidx.at[j, pl.ds(u * _QS, _QS)]],
                                bufs.at[u], sems[u]) for u in range(_NSUB)]
        for u in range(_NSUB):
            cps[u].wait()
            pltpu.sync_copy(bufs.at[u], acc_sh.at[sidx.at[u]], add=True)
        return carry

    lax.fori_loop(1, K, j_step, 0)

    pltpu.sync_copy(acc_sh.at[pl.ds(sbase, _QPW)], out_hbm.at[pl.ds(base, _QPW)])


def _sc_agg(z, nbr, width):
    mesh = plsc.VectorSubcoreMesh(core_axis_name="c", subcore_axis_name="s")
    fn = pl.kernel(
        _agg_body,
        out_type=jax.ShapeDtypeStruct((NP, width), jnp.float32),
        mesh=mesh,
        scratch_types=[
            pltpu.VMEM((KPAD, _QPW), jnp.int32),
            pltpu.VMEM((_NSUB, _QS), jnp.int32),
            pltpu.VMEM((_NSUB, _QS, width), jnp.float32),
            pltpu.VMEM_SHARED((_NS * _QPW, width), jnp.float32),
            pltpu.SemaphoreType.DMA,
            pltpu.SemaphoreType.DMA,
            pltpu.SemaphoreType.DMA,
            pltpu.SemaphoreType.DMA,
        ],
        compiler_params=pltpu.CompilerParams(use_tc_tiling_on_sc=False),
    )
    return fn(nbr, z)


_ROWS = 1000


def _layer_body(has_agg, x_ref, *refs):
    if has_agg:
        g_ref, a_ref, c_ref, w_ref, b_ref, z_ref, s_ref = refs
        x = x_ref[...] + g_ref[...]
    else:
        a_ref, c_ref, w_ref, b_ref, z_ref, s_ref = refs
        x = x_ref[...]
    x = a_ref[0:1, :] * x + c_ref[0:1, :]
    z = jnp.dot(x.astype(jnp.bfloat16), w_ref[...],
                preferred_element_type=jnp.float32)
    z = z + b_ref[0:1, :]
    z = jnp.where(z >= 0, z, jnp.float32(0.33) * z)
    z_ref[...] = z
    cout = z.shape[1]
    s1 = jnp.sum(z, axis=0)[None, :]
    s2 = jnp.sum(z * z, axis=0)[None, :]
    r8 = lax.broadcasted_iota(jnp.int32, (8, cout), 0)
    s8 = jnp.where(r8 == 0, s1, jnp.where(r8 == 1, s2, jnp.float32(0.0)))
    i = pl.program_id(0)

    @pl.when(i == 0)
    def _init():
        s_ref[...] = jnp.zeros((8, cout), jnp.float32)

    s_ref[...] += s8


def _mlp_layer(x, agg, a, c, W, b):
    cin, cout = W.shape
    a8 = jnp.broadcast_to(a[None, :], (8, cin))
    c8 = jnp.broadcast_to(c[None, :], (8, cin))
    b8 = jnp.broadcast_to(b[None, :], (8, cout))
    wb = W.astype(jnp.bfloat16)
    ins = [x] + ([agg] if agg is not None else []) + [a8, c8, wb, b8]
    in_specs = [pl.BlockSpec((_ROWS, cin), lambda i: (i, 0))]
    if agg is not None:
        in_specs.append(pl.BlockSpec((_ROWS, cin), lambda i: (i, 0)))
    in_specs += [
        pl.BlockSpec((8, cin), lambda i: (0, 0)),
        pl.BlockSpec((8, cin), lambda i: (0, 0)),
        pl.BlockSpec((cin, cout), lambda i: (0, 0)),
        pl.BlockSpec((8, cout), lambda i: (0, 0)),
    ]
    return pl.pallas_call(
        functools.partial(_layer_body, agg is not None),
        grid=(N // _ROWS,),
        in_specs=in_specs,
        out_specs=[
            pl.BlockSpec((_ROWS, cout), lambda i: (i, 0)),
            pl.BlockSpec((8, cout), lambda i: (0, 0)),
        ],
        out_shape=[
            jax.ShapeDtypeStruct((N, cout), jnp.float32),
            jax.ShapeDtypeStruct((8, cout), jnp.float32),
        ],
    )(*ins)


def _final_body(x_ref, a_ref, c_ref, w_ref, b_ref, o_ref):
    x = a_ref[0:1, :] * x_ref[...] + c_ref[0:1, :]
    z = jnp.dot(x.astype(jnp.bfloat16), w_ref[...],
                preferred_element_type=jnp.float32)
    o_ref[...] = z + b_ref[0:1, :]


def _final_layer(x, a, c, W, b):
    cin, cout = W.shape
    a8 = jnp.broadcast_to(a[None, :], (8, cin))
    c8 = jnp.broadcast_to(c[None, :], (8, cin))
    b8 = jnp.broadcast_to(b[None, :], (8, cout))
    return pl.pallas_call(
        _final_body,
        grid=(N // _ROWS,),
        in_specs=[
            pl.BlockSpec((_ROWS, cin), lambda i: (i, 0)),
            pl.BlockSpec((8, cin), lambda i: (0, 0)),
            pl.BlockSpec((8, cin), lambda i: (0, 0)),
            pl.BlockSpec((cin, cout), lambda i: (0, 0)),
            pl.BlockSpec((8, cout), lambda i: (0, 0)),
        ],
        out_specs=pl.BlockSpec((_ROWS, cout), lambda i: (i, 0)),
        out_shape=jax.ShapeDtypeStruct((N, cout), jnp.float32),
    )(x, a8, c8, W.astype(jnp.bfloat16), b8)


def _stats_to_affine(sums, g, be):
    m = sums[0] / N
    v = sums[1] / N - m * m
    a = g / jnp.sqrt(v + 1e-5)
    return a, be - m * a


def kernel(input, params):
    pc = input
    coords = pc[:, 0:3]
    sq = jnp.sum(coords * coords, axis=1)

    cpad = jnp.pad(coords, ((0, NP - N), (0, 0)))
    qb = jnp.pad(cpad, ((0, 0), (0, 5))).astype(jnp.bfloat16)
    cb = jnp.pad(cpad.T, ((0, 5), (0, 0))).astype(jnp.bfloat16)
    sqq = jnp.pad(sq, (0, NP - N))[:, None]
    sqc8 = jnp.broadcast_to(
        jnp.pad(sq, (0, NP - N), constant_values=1e30)[None, :], (8, NP))

    nbr = _knn(qb, cb, sqq, sqc8)
    nbr_flat = nbr.T.reshape(-1)

    nc = (coords - 384.0) / 384.0
    x0 = jnp.concatenate([nc, pc[:, 4:5], jnp.zeros((N, 12), jnp.float32)], axis=1)

    z = x0
    a = jnp.ones((16,), jnp.float32)
    c = jnp.zeros((16,), jnp.float32)

    for name in ("gin1", "gin2", "gin3"):
        layers = params[name]
        aggz = _sc_agg(z, nbr_flat, z.shape[1])[:N]
        for li, (W, b, g, be) in enumerate(layers):
            if name == "gin1" and li == 0:
                W = jnp.pad(W, ((0, 12), (0, 0)))
            if li == 0:
                z, sums = _mlp_layer(z, aggz, a, (1.0 + K) * c, W, b)
            else:
                z, sums = _mlp_layer(z, None, a, c, W, b)
            a, c = _stats_to_affine(sums, g, be)

    for (W, b, g, be) in params["mlp3"]:
        z, sums = _mlp_layer(z, None, a, c, W, b)
        a, c = _stats_to_affine(sums, g, be)

    Wfin, bfin = params["final"]
    return _final_layer(z, a, c, Wfin, bfin)


# speedup vs baseline: 2.5734x; 1.2018x over previous
import functools

import jax
import jax.numpy as jnp
from jax import lax
from jax.experimental import pallas as pl
from jax.experimental.pallas import tpu as pltpu
from jax.experimental.pallas import tpu_sc as plsc

N = 10000
NP = 10240
K = 10
KPAD = 16


_KNN_R = 256


_SUBR = 128
_NG = NP // 128


def _knn_body(q_ref, c_ref, sqq_ref, sqc_ref, o_ref, d2_ref):
    pid = pl.program_id(0)
    lane = lax.broadcasted_iota(jnp.int32, (_SUBR, 128), 1)
    BIGF = jnp.float32(4e30)
    BIGI = jnp.int32(NP)

    dot = jnp.dot(q_ref[...], c_ref[...], preferred_element_type=jnp.float32)
    colsf = lax.broadcasted_iota(jnp.int32, (_KNN_R, NP), 1)
    rowsf = lax.broadcasted_iota(jnp.int32, (_KNN_R, NP), 0) + pid * _KNN_R
    d2all = (sqq_ref[...] + sqc_ref[0:1, :]) - 2.0 * dot
    d2_ref[...] = jnp.where(colsf == rowsf, BIGF, d2all)

    def sub_body(s, carry):
        V1 = V2 = V3 = V4 = jnp.full((_SUBR, 128), BIGF, jnp.float32)
        I1 = I2 = I3 = I4 = jnp.full((_SUBR, 128), BIGI, jnp.int32)
        for g in range(_NG):
            d2 = d2_ref[pl.ds(s * _SUBR, _SUBR), g * 128:(g + 1) * 128]
            colg = lane + (g * 128)
            b1 = d2 < V1
            b2 = d2 < V2
            b3 = d2 < V3
            b4 = d2 < V4
            V4 = jnp.where(b4, jnp.where(b3, V3, d2), V4)
            I4 = jnp.where(b4, jnp.where(b3, I3, colg), I4)
            V3 = jnp.where(b3, jnp.where(b2, V2, d2), V3)
            I3 = jnp.where(b3, jnp.where(b2, I2, colg), I3)
            V2 = jnp.where(b2, jnp.where(b1, V1, d2), V2)
            I2 = jnp.where(b2, jnp.where(b1, I1, colg), I2)
            V1 = jnp.where(b1, d2, V1)
            I1 = jnp.where(b1, colg, I1)
        VV = jnp.concatenate([V1, V2, V3, V4], axis=1)
        II = jnp.concatenate([I1, I2, I3, I4], axis=1)
        m = idx = None
        for t in range(K):
            m = jnp.min(VV, axis=1)
            idx = jnp.min(jnp.where(VV == m[:, None], II, BIGI), axis=1)
            o_ref[pl.ds(s * _SUBR, _SUBR), pl.ds(t, 1)] = idx[:, None]
            VV = jnp.where((VV == m[:, None]) & (II == idx[:, None]), BIGF, VV)
        bad = (V4 < m[:, None]) | ((V4 == m[:, None]) & (I4 <= idx[:, None]))
        flag = jnp.any(bad)

        @pl.when(flag)
        def _fallback():
            d2f = d2_ref[pl.ds(s * _SUBR, _SUBR), :]
            colf = lax.broadcasted_iota(jnp.int32, (_SUBR, NP), 1)
            for t in range(K):
                mf = jnp.min(d2f, axis=1)
                idxf = jnp.min(jnp.where(d2f == mf[:, None], colf, BIGI),
                               axis=1)
                o_ref[pl.ds(s * _SUBR, _SUBR), pl.ds(t, 1)] = idxf[:, None]
                d2f = jnp.where(colf == idxf[:, None], BIGF, d2f)

        return carry

    lax.fori_loop(0, _KNN_R // _SUBR, sub_body, 0)


def _knn(qb, cb, sqq, sqc8):
    return pl.pallas_call(
        _knn_body,
        grid=(NP // _KNN_R,),
        in_specs=[
            pl.BlockSpec((_KNN_R, 8), lambda i: (i, 0)),
            pl.BlockSpec((8, NP), lambda i: (0, 0)),
            pl.BlockSpec((_KNN_R, 1), lambda i: (i, 0)),
            pl.BlockSpec((8, NP), lambda i: (0, 0)),
        ],
        out_specs=pl.BlockSpec((_KNN_R, KPAD), lambda i: (i, 0)),
        out_shape=jax.ShapeDtypeStruct((NP, KPAD), jnp.int32),
        scratch_shapes=[pltpu.VMEM((_KNN_R, NP), jnp.float32)],
    )(qb, cb, sqq, sqc8)


_NC, _NS = 2, 16
_NW = _NC * _NS
_QPW = NP // _NW
_QS = 80
_NSUB = _QPW // _QS


def _agg_body(nbr_hbm, z_hbm, out_hbm, gidx, sidx, bufs, acc_sh,
              s0, s1, s2, s3):
    cid = lax.axis_index("c")
    sid = lax.axis_index("s")
    wid = cid * _NS + sid
    base = wid * _QPW
    sbase = sid * _QPW
    sems = (s0, s1, s2, s3)

    for j in range(K):
        pltpu.sync_copy(nbr_hbm.at[pl.ds(j * NP + base, _QPW)], gidx.at[j])

    for u in range(_NSUB):
        for t in range(_QS // 16):
            sidx[u, pl.ds(t * 16, 16)] = (
                lax.iota(jnp.int32, 16) + (sbase + u * _QS + t * 16))

    cps = [pltpu.async_copy(z_hbm.at[gidx.at[0, pl.ds(u * _QS, _QS)]],
                            bufs.at[u], sems[u]) for u in range(_NSUB)]
    for u in range(_NSUB):
        cps[u].wait()
        pltpu.sync_copy(bufs.at[u], acc_sh.at[pl.ds(sbase + u * _QS, _QS)])

    def j_step(j, carry):
        cps = [pltpu.async_copy(z_hbm.at[gidx.at[j, pl.ds(u * _QS, _QS)]],
                                bufs.at[u], sems[u]) for u in range(_NSUB)]
        for u in range(_NSUB):
            cps[u].wait()
            pltpu.sync_copy(bufs.at[u], acc_sh.at[sidx.at[u]], add=True)
        return carry

    lax.fori_loop(1, K, j_step, 0)

    pltpu.sync_copy(acc_sh.at[pl.ds(sbase, _QPW)], out_hbm.at[pl.ds(base, _QPW)])


def _sc_agg(z, nbr, width):
    mesh = plsc.VectorSubcoreMesh(core_axis_name="c", subcore_axis_name="s")
    fn = pl.kernel(
        _agg_body,
        out_type=jax.ShapeDtypeStruct((NP, width), jnp.float32),
        mesh=mesh,
        scratch_types=[
            pltpu.VMEM((KPAD, _QPW), jnp.int32),
            pltpu.VMEM((_NSUB, _QS), jnp.int32),
            pltpu.VMEM((_NSUB, _QS, width), jnp.float32),
            pltpu.VMEM_SHARED((_NS * _QPW, width), jnp.float32),
            pltpu.SemaphoreType.DMA,
            pltpu.SemaphoreType.DMA,
            pltpu.SemaphoreType.DMA,
            pltpu.SemaphoreType.DMA,
        ],
        compiler_params=pltpu.CompilerParams(use_tc_tiling_on_sc=False),
    )
    return fn(nbr, z)


_ROWS = 1000


def _layer_body(has_agg, x_ref, *refs):
    if has_agg:
        g_ref, a_ref, c_ref, w_ref, b_ref, z_ref, s_ref = refs
        x = x_ref[...] + g_ref[...]
    else:
        a_ref, c_ref, w_ref, b_ref, z_ref, s_ref = refs
        x = x_ref[...]
    x = a_ref[0:1, :] * x + c_ref[0:1, :]
    z = jnp.dot(x.astype(jnp.bfloat16), w_ref[...],
                preferred_element_type=jnp.float32)
    z = z + b_ref[0:1, :]
    z = jnp.where(z >= 0, z, jnp.float32(0.33) * z)
    z_ref[...] = z
    cout = z.shape[1]
    s1 = jnp.sum(z, axis=0)[None, :]
    s2 = jnp.sum(z * z, axis=0)[None, :]
    r8 = lax.broadcasted_iota(jnp.int32, (8, cout), 0)
    s8 = jnp.where(r8 == 0, s1, jnp.where(r8 == 1, s2, jnp.float32(0.0)))
    i = pl.program_id(0)

    @pl.when(i == 0)
    def _init():
        s_ref[...] = jnp.zeros((8, cout), jnp.float32)

    s_ref[...] += s8


def _mlp_layer(x, agg, a, c, W, b):
    cin, cout = W.shape
    a8 = jnp.broadcast_to(a[None, :], (8, cin))
    c8 = jnp.broadcast_to(c[None, :], (8, cin))
    b8 = jnp.broadcast_to(b[None, :], (8, cout))
    wb = W.astype(jnp.bfloat16)
    ins = [x] + ([agg] if agg is not None else []) + [a8, c8, wb, b8]
    in_specs = [pl.BlockSpec((_ROWS, cin), lambda i: (i, 0))]
    if agg is not None:
        in_specs.append(pl.BlockSpec((_ROWS, cin), lambda i: (i, 0)))
    in_specs += [
        pl.BlockSpec((8, cin), lambda i: (0, 0)),
        pl.BlockSpec((8, cin), lambda i: (0, 0)),
        pl.BlockSpec((cin, cout), lambda i: (0, 0)),
        pl.BlockSpec((8, cout), lambda i: (0, 0)),
    ]
    return pl.pallas_call(
        functools.partial(_layer_body, agg is not None),
        grid=(N // _ROWS,),
        in_specs=in_specs,
        out_specs=[
            pl.BlockSpec((_ROWS, cout), lambda i: (i, 0)),
            pl.BlockSpec((8, cout), lambda i: (0, 0)),
        ],
        out_shape=[
            jax.ShapeDtypeStruct((N, cout), jnp.float32),
            jax.ShapeDtypeStruct((8, cout), jnp.float32),
        ],
    )(*ins)


def _final_body(x_ref, a_ref, c_ref, w_ref, b_ref, o_ref):
    x = a_ref[0:1, :] * x_ref[...] + c_ref[0:1, :]
    z = jnp.dot(x.astype(jnp.bfloat16), w_ref[...],
                preferred_element_type=jnp.float32)
    o_ref[...] = z + b_ref[0:1, :]


def _final_layer(x, a, c, W, b):
    cin, cout = W.shape
    a8 = jnp.broadcast_to(a[None, :], (8, cin))
    c8 = jnp.broadcast_to(c[None, :], (8, cin))
    b8 = jnp.broadcast_to(b[None, :], (8, cout))
    return pl.pallas_call(
        _final_body,
        grid=(N // _ROWS,),
        in_specs=[
            pl.BlockSpec((_ROWS, cin), lambda i: (i, 0)),
            pl.BlockSpec((8, cin), lambda i: (0, 0)),
            pl.BlockSpec((8, cin), lambda i: (0, 0)),
            pl.BlockSpec((cin, cout), lambda i: (0, 0)),
            pl.BlockSpec((8, cout), lambda i: (0, 0)),
        ],
        out_specs=pl.BlockSpec((_ROWS, cout), lambda i: (i, 0)),
        out_shape=jax.ShapeDtypeStruct((N, cout), jnp.float32),
    )(x, a8, c8, W.astype(jnp.bfloat16), b8)


def _stats_to_affine(sums, g, be):
    m = sums[0] / N
    v = sums[1] / N - m * m
    a = g / jnp.sqrt(v + 1e-5)
    return a, be - m * a


def kernel(input, params):
    pc = input
    coords = pc[:, 0:3]
    sq = jnp.sum(coords * coords, axis=1)

    cpad = jnp.pad(coords, ((0, NP - N), (0, 0)))
    qb = jnp.pad(cpad, ((0, 0), (0, 5))).astype(jnp.bfloat16)
    cb = jnp.pad(cpad.T, ((0, 5), (0, 0))).astype(jnp.bfloat16)
    sqq = jnp.pad(sq, (0, NP - N))[:, None]
    sqc8 = jnp.broadcast_to(
        jnp.pad(sq, (0, NP - N), constant_values=1e30)[None, :], (8, NP))

    nbr = _knn(qb, cb, sqq, sqc8)
    nbr_flat = nbr.T.reshape(-1)

    nc = (coords - 384.0) / 384.0
    x0 = jnp.concatenate([nc, pc[:, 4:5], jnp.zeros((N, 12), jnp.float32)], axis=1)

    z = x0
    a = jnp.ones((16,), jnp.float32)
    c = jnp.zeros((16,), jnp.float32)

    for name in ("gin1", "gin2", "gin3"):
        layers = params[name]
        aggz = _sc_agg(z, nbr_flat, z.shape[1])[:N]
        for li, (W, b, g, be) in enumerate(layers):
            if name == "gin1" and li == 0:
                W = jnp.pad(W, ((0, 12), (0, 0)))
            if li == 0:
                z, sums = _mlp_layer(z, aggz, a, (1.0 + K) * c, W, b)
            else:
                z, sums = _mlp_layer(z, None, a, c, W, b)
            a, c = _stats_to_affine(sums, g, be)

    for (W, b, g, be) in params["mlp3"]:
        z, sums = _mlp_layer(z, None, a, c, W, b)
        a, c = _stats_to_affine(sums, g, be)

    Wfin, bfin = params["final"]
    return _final_layer(z, a, c, Wfin, bfin)
